# Initial kernel scaffold; baseline (speedup 1.0000x reference)
import jax
import jax.numpy as jnp
from jax.experimental import pallas as pl


def kernel(x, router_w, w1, w2):
    raise NotImplementedError("write your pallas kernel here")


# speedup vs baseline: 2.7141x; 2.7141x over previous
import functools

import jax
import jax.numpy as jnp
from jax.experimental import pallas as pl
from jax.experimental.pallas import tpu as pltpu

_INTERPRET = False


def _moe_dense_body(x_ref, rw_ref, w1_ref, w2_ref, out_ref, loss_ref, acc_ref,
                    *, n_exp, width, n_tok):
    j = pl.program_id(0)
    nt = pl.num_programs(0)
    x = x_ref[...]
    rw = rw_ref[...]
    gt = x.shape[0]
    d = x.shape[1]

    logits = jax.lax.dot_general(x, rw, (((1,), (1,)), ((), ())),
                                 preferred_element_type=jnp.float32)
    probs = jax.nn.sigmoid(logits)
    lanes = jax.lax.broadcasted_iota(jnp.int32, probs.shape, 1)
    v1 = jnp.max(probs, axis=1, keepdims=True)
    i1 = jnp.min(jnp.where(probs == v1, lanes, n_exp), axis=1, keepdims=True)
    masked = jnp.where(lanes == i1, -jnp.inf, probs)
    v2 = jnp.max(masked, axis=1, keepdims=True)
    i2 = jnp.min(jnp.where(masked == v2, lanes, n_exp), axis=1, keepdims=True)
    ssum = v1 + v2 + 1e-20
    oh1 = (lanes == i1).astype(jnp.float32)
    oh2 = (lanes == i2).astype(jnp.float32)
    combine = (oh1 * v1 + oh2 * v2) / ssum

    m = jnp.max(logits, axis=1, keepdims=True)
    lse = m + jnp.log(jnp.sum(jnp.exp(logits - m), axis=1, keepdims=True))
    z_part = jnp.sum(lse * lse)

    @pl.when(j == 0)
    def _():
        acc_ref[...] = jnp.zeros_like(acc_ref)

    acc_ref[0:1, :] += jnp.sum(probs, axis=0, keepdims=True)
    acc_ref[1:2, :] += jnp.sum(oh1 + oh2, axis=0, keepdims=True)
    acc_ref[2:3, :] += jnp.full((1, n_exp), z_part, jnp.float32)

    acc = jnp.zeros((gt, d), jnp.float32)
    for e in range(n_exp):
        w1e = w1_ref[:, e * width:(e + 1) * width]
        w2e = w2_ref[e * width:(e + 1) * width, :]
        h = jnp.dot(x, w1e, preferred_element_type=jnp.float32)
        a = jnp.square(jnp.maximum(h, 0.0))
        y = jnp.dot(a, w2e, preferred_element_type=jnp.float32)
        acc += combine[:, e:e + 1] * y
    out_ref[...] = acc

    @pl.when(j == nt - 1)
    def _():
        p_i = acc_ref[0:1, :] / n_tok
        cnt = acc_ref[1:2, :]
        f_i = cnt / (2.0 * n_tok)
        z = acc_ref[2, 0] / n_tok
        lb = n_exp * jnp.sum(f_i * p_i)
        closs = jnp.sum(acc_ref[0:1, :]) / n_tok
        loss_ref[0:1, :] = f_i
        loss_ref[1:2, :] = jnp.full((1, n_exp), z, jnp.float32)
        loss_ref[2:3, :] = jnp.full((1, n_exp), lb, jnp.float32)
        loss_ref[3:4, :] = jnp.full((1, n_exp), closs, jnp.float32)
        loss_ref[4:8, :] = jnp.zeros((4, n_exp), jnp.float32)


def kernel(x, router_w, w1, w2):
    b, s, d = x.shape
    n_exp, _ = router_w.shape
    total_w = w1.shape[1]
    width = total_w // n_exp
    t = b * s
    gt = 256
    nt = t // gt

    x_flat = x.reshape(t, d)
    body = functools.partial(_moe_dense_body, n_exp=n_exp, width=width, n_tok=t)
    out_flat, lossbuf = pl.pallas_call(
        body,
        grid=(nt,),
        in_specs=[
            pl.BlockSpec((gt, d), lambda j: (j, 0)),
            pl.BlockSpec((n_exp, d), lambda j: (0, 0)),
            pl.BlockSpec((d, total_w), lambda j: (0, 0)),
            pl.BlockSpec((total_w, d), lambda j: (0, 0)),
        ],
        out_specs=[
            pl.BlockSpec((gt, d), lambda j: (j, 0)),
            pl.BlockSpec((8, n_exp), lambda j: (0, 0)),
        ],
        out_shape=[
            jax.ShapeDtypeStruct((t, d), jnp.float32),
            jax.ShapeDtypeStruct((8, n_exp), jnp.float32),
        ],
        scratch_shapes=[pltpu.VMEM((8, n_exp), jnp.float32)],
        interpret=_INTERPRET,
    )(x_flat, router_w, w1, w2)

    output = out_flat.reshape(b, s, d)
    f_i = lossbuf[0]
    z = lossbuf[1, 0]
    lb = lossbuf[2, 0]
    cl = lossbuf[3, 0]
    return (output, z, lb, cl, f_i)


# speedup vs baseline: 2.7623x; 1.0178x over previous
import functools

import jax
import jax.numpy as jnp
from jax.experimental import pallas as pl
from jax.experimental.pallas import tpu as pltpu

_INTERPRET = False


def _moe_dense_body(x_ref, rw_ref, w1_ref, w2_ref, out_ref, loss_ref, acc_ref,
                    *, n_exp, width, n_tok):
    j = pl.program_id(0)
    nt = pl.num_programs(0)
    x = x_ref[...]
    rw = rw_ref[...]
    gt = x.shape[0]
    d = x.shape[1]

    logits = jax.lax.dot_general(x, rw, (((1,), (1,)), ((), ())),
                                 preferred_element_type=jnp.float32)
    probs = jax.nn.sigmoid(logits)
    lanes = jax.lax.broadcasted_iota(jnp.int32, probs.shape, 1)
    v1 = jnp.max(probs, axis=1, keepdims=True)
    i1 = jnp.min(jnp.where(probs == v1, lanes, n_exp), axis=1, keepdims=True)
    masked = jnp.where(lanes == i1, -jnp.inf, probs)
    v2 = jnp.max(masked, axis=1, keepdims=True)
    i2 = jnp.min(jnp.where(masked == v2, lanes, n_exp), axis=1, keepdims=True)
    ssum = v1 + v2 + 1e-20
    oh1 = (lanes == i1).astype(jnp.float32)
    oh2 = (lanes == i2).astype(jnp.float32)
    combine = (oh1 * v1 + oh2 * v2) / ssum

    m = jnp.max(logits, axis=1, keepdims=True)
    lse = m + jnp.log(jnp.sum(jnp.exp(logits - m), axis=1, keepdims=True))
    z_part = jnp.sum(lse * lse)

    @pl.when(j == 0)
    def _():
        acc_ref[...] = jnp.zeros_like(acc_ref)

    acc_ref[0:1, :] += jnp.sum(probs, axis=0, keepdims=True)
    acc_ref[1:2, :] += jnp.sum(oh1 + oh2, axis=0, keepdims=True)
    acc_ref[2:3, :] += jnp.full((1, n_exp), z_part, jnp.float32)

    h = jnp.dot(x, w1_ref[...], preferred_element_type=jnp.float32)
    a = jnp.square(jnp.maximum(h, 0.0))
    lane_e = jax.lax.broadcasted_iota(jnp.int32, a.shape, 1) // width
    c_exp = jnp.zeros_like(a)
    for e in range(n_exp):
        c_exp = jnp.where(lane_e == e, combine[:, e:e + 1], c_exp)
    a = (a * c_exp).astype(jnp.bfloat16)
    out_ref[...] = jnp.dot(a, w2_ref[...], preferred_element_type=jnp.float32)

    @pl.when(j == nt - 1)
    def _():
        p_i = acc_ref[0:1, :] / n_tok
        cnt = acc_ref[1:2, :]
        f_i = cnt / (2.0 * n_tok)
        z = acc_ref[2, 0] / n_tok
        lb = n_exp * jnp.sum(f_i * p_i)
        closs = jnp.sum(acc_ref[0:1, :]) / n_tok
        loss_ref[0:1, :] = f_i
        loss_ref[1:2, :] = jnp.full((1, n_exp), z, jnp.float32)
        loss_ref[2:3, :] = jnp.full((1, n_exp), lb, jnp.float32)
        loss_ref[3:4, :] = jnp.full((1, n_exp), closs, jnp.float32)
        loss_ref[4:8, :] = jnp.zeros((4, n_exp), jnp.float32)


def kernel(x, router_w, w1, w2):
    b, s, d = x.shape
    n_exp, _ = router_w.shape
    total_w = w1.shape[1]
    width = total_w // n_exp
    t = b * s
    gt = 256
    nt = t // gt

    x_flat = x.reshape(t, d)
    w2_bf = w2.astype(jnp.bfloat16)
    body = functools.partial(_moe_dense_body, n_exp=n_exp, width=width, n_tok=t)
    out_flat, lossbuf = pl.pallas_call(
        body,
        grid=(nt,),
        in_specs=[
            pl.BlockSpec((gt, d), lambda j: (j, 0)),
            pl.BlockSpec((n_exp, d), lambda j: (0, 0)),
            pl.BlockSpec((d, total_w), lambda j: (0, 0)),
            pl.BlockSpec((total_w, d), lambda j: (0, 0)),
        ],
        out_specs=[
            pl.BlockSpec((gt, d), lambda j: (j, 0)),
            pl.BlockSpec((8, n_exp), lambda j: (0, 0)),
        ],
        out_shape=[
            jax.ShapeDtypeStruct((t, d), jnp.float32),
            jax.ShapeDtypeStruct((8, n_exp), jnp.float32),
        ],
        scratch_shapes=[pltpu.VMEM((8, n_exp), jnp.float32)],
        interpret=_INTERPRET,
    )(x_flat, router_w, w1, w2_bf)

    output = out_flat.reshape(b, s, d)
    f_i = lossbuf[0]
    z = lossbuf[1, 0]
    lb = lossbuf[2, 0]
    cl = lossbuf[3, 0]
    return (output, z, lb, cl, f_i)


# speedup vs baseline: 3.1549x; 1.1421x over previous
import functools

import jax
import jax.numpy as jnp
from jax.experimental import pallas as pl
from jax.experimental.pallas import tpu as pltpu

_INTERPRET = False


def _moe_dense_body(x_ref, rw_ref, w1_ref, w2_ref, out_ref, loss_ref, acc_ref,
                    *, n_exp, width, n_tok):
    j = pl.program_id(0)
    nt = pl.num_programs(0)
    x = x_ref[...]
    rw = rw_ref[...]
    gt = x.shape[0]
    d = x.shape[1]

    logits = jax.lax.dot_general(x, rw, (((1,), (1,)), ((), ())),
                                 preferred_element_type=jnp.float32)
    probs = jax.nn.sigmoid(logits)
    lanes = jax.lax.broadcasted_iota(jnp.int32, probs.shape, 1)
    v1 = jnp.max(probs, axis=1, keepdims=True)
    i1 = jnp.min(jnp.where(probs == v1, lanes, n_exp), axis=1, keepdims=True)
    masked = jnp.where(lanes == i1, -jnp.inf, probs)
    v2 = jnp.max(masked, axis=1, keepdims=True)
    i2 = jnp.min(jnp.where(masked == v2, lanes, n_exp), axis=1, keepdims=True)
    ssum = v1 + v2 + 1e-20
    oh1 = (lanes == i1).astype(jnp.float32)
    oh2 = (lanes == i2).astype(jnp.float32)
    combine = (oh1 * v1 + oh2 * v2) / ssum

    m = jnp.max(logits, axis=1, keepdims=True)
    lse = m + jnp.log(jnp.sum(jnp.exp(logits - m), axis=1, keepdims=True))
    z_part = jnp.sum(lse * lse)

    @pl.when(j == 0)
    def _():
        acc_ref[...] = jnp.zeros_like(acc_ref)

    acc_ref[0:1, :] += jnp.sum(probs, axis=0, keepdims=True)
    acc_ref[1:2, :] += jnp.sum(oh1 + oh2, axis=0, keepdims=True)
    acc_ref[2:3, :] += jnp.full((1, n_exp), z_part, jnp.float32)

    h = jnp.dot(x, w1_ref[...], preferred_element_type=jnp.float32)
    a = jnp.square(jnp.maximum(h, 0.0))
    lane_e = jax.lax.broadcasted_iota(jnp.int32, a.shape, 1) // width
    c_exp = jnp.zeros_like(a)
    for e in range(n_exp):
        c_exp = jnp.where(lane_e == e, combine[:, e:e + 1], c_exp)
    a = a * c_exp
    out_ref[...] = jnp.dot(a, w2_ref[...], preferred_element_type=jnp.float32)

    @pl.when(j == nt - 1)
    def _():
        p_i = acc_ref[0:1, :] / n_tok
        cnt = acc_ref[1:2, :]
        f_i = cnt / (2.0 * n_tok)
        z = acc_ref[2, 0] / n_tok
        lb = n_exp * jnp.sum(f_i * p_i)
        closs = jnp.sum(acc_ref[0:1, :]) / n_tok
        loss_ref[0:1, :] = f_i
        loss_ref[1:2, :] = jnp.full((1, n_exp), z, jnp.float32)
        loss_ref[2:3, :] = jnp.full((1, n_exp), lb, jnp.float32)
        loss_ref[3:4, :] = jnp.full((1, n_exp), closs, jnp.float32)
        loss_ref[4:8, :] = jnp.zeros((4, n_exp), jnp.float32)


def kernel(x, router_w, w1, w2):
    b, s, d = x.shape
    n_exp, _ = router_w.shape
    total_w = w1.shape[1]
    width = total_w // n_exp
    t = b * s
    gt = 256
    nt = t // gt

    x_flat = x.reshape(t, d)
    body = functools.partial(_moe_dense_body, n_exp=n_exp, width=width, n_tok=t)
    out_flat, lossbuf = pl.pallas_call(
        body,
        grid=(nt,),
        in_specs=[
            pl.BlockSpec((gt, d), lambda j: (j, 0)),
            pl.BlockSpec((n_exp, d), lambda j: (0, 0)),
            pl.BlockSpec((d, total_w), lambda j: (0, 0)),
            pl.BlockSpec((total_w, d), lambda j: (0, 0)),
        ],
        out_specs=[
            pl.BlockSpec((gt, d), lambda j: (j, 0)),
            pl.BlockSpec((8, n_exp), lambda j: (0, 0)),
        ],
        out_shape=[
            jax.ShapeDtypeStruct((t, d), jnp.float32),
            jax.ShapeDtypeStruct((8, n_exp), jnp.float32),
        ],
        scratch_shapes=[pltpu.VMEM((8, n_exp), jnp.float32)],
        interpret=_INTERPRET,
    )(x_flat, router_w, w1, w2)

    output = out_flat.reshape(b, s, d)
    f_i = lossbuf[0]
    z = lossbuf[1, 0]
    lb = lossbuf[2, 0]
    cl = lossbuf[3, 0]
    return (output, z, lb, cl, f_i)
